# baseline (speedup 1.0000x reference)
import functools

import jax
import jax.numpy as jnp
from jax import lax
from jax.experimental import pallas as pl
from jax.experimental.pallas import tpu as pltpu
from jax.experimental.pallas import tpu_sc as plsc

VOCAB = 49408
NUM_POS = 77
EMBED = 768
BATCH = 1024

L = 16
NC = 2
NS = 16
NW = NC * NS
BATCH_PER_W = BATCH // NW
NBUF = 7
CHUNK = 11
DEPTH = NBUF - 2
NCH_W = BATCH_PER_W * NBUF


def _emb_body(idx_hbm, tok_hbm, pos_hbm, out_hbm, idx_v, pos_v, *rest):
    bufs = rest[:NBUF]
    gsem = rest[NBUF:2 * NBUF]
    ssem = rest[2 * NBUF:3 * NBUF]

    wid = lax.axis_index("s") * NC + lax.axis_index("c")
    lb0 = wid * BATCH_PER_W

    pltpu.sync_copy(idx_hbm.at[wid], idx_v)
    pltpu.sync_copy(pos_hbm, pos_v)

    def gather(lb, j):
        src = tok_hbm.at[idx_v.at[lb * NBUF + j]]
        return pltpu.make_async_copy(src, bufs[j], gsem[j])

    def scatter(lb, j):
        row = (lb0 + lb) * NUM_POS + j * CHUNK
        dst = out_hbm.at[pl.ds(row, CHUNK)]
        return pltpu.make_async_copy(bufs[j], dst, ssem[j])

    for j in range(DEPTH):
        gather(0, j).start()

    def outer(lb, carry):
        for j in range(NBUF):
            cglob = lb * NBUF + j
            jq = (j + DEPTH) % NBUF
            lbq = lb + (0 if j + DEPTH < NBUF else 1)

            @pl.when(cglob + DEPTH < NCH_W)
            def _():
                @pl.when(cglob >= 2)
                def _():
                    scatter(lbq - 1, jq).wait()
                gather(lbq, jq).start()

            gather(lb, j).wait()

            def row_add(r, carry2):
                for k in range(EMBED // L):
                    sl = pl.ds(k * L, L)
                    plsc.addupdate(bufs[j].at[r, sl],
                                   pos_v[j * CHUNK + r, sl])
                return carry2

            lax.fori_loop(0, CHUNK, row_add, 0, unroll=False)

            scatter(lb, j).start()
        return carry

    lax.fori_loop(0, BATCH_PER_W, outer, 0, unroll=False)

    for j in range(NBUF):
        scatter(BATCH_PER_W - 1, j).wait()


@jax.jit
def _emb_call(idx, token_table, position_table):
    info = plsc.get_sparse_core_info()
    assert info.num_cores == NC and info.num_subcores == NS

    mesh = plsc.VectorSubcoreMesh(core_axis_name="c", subcore_axis_name="s",
                                  num_cores=NC)
    run = functools.partial(
        pl.kernel,
        mesh=mesh,
        compiler_params=pltpu.CompilerParams(use_tc_tiling_on_sc=False),
        out_type=jax.ShapeDtypeStruct((BATCH * NUM_POS, EMBED), jnp.float32),
        scratch_types=(
            [pltpu.VMEM((NCH_W, CHUNK), jnp.int32),
             pltpu.VMEM((NUM_POS, EMBED), jnp.float32)]
            + [pltpu.VMEM((CHUNK, EMBED), jnp.float32)] * NBUF
            + [pltpu.SemaphoreType.DMA] * (2 * NBUF)
        ),
    )(_emb_body)
    return run(idx, token_table, position_table)


def kernel(input_tokens, token_table, position_table):
    idx = input_tokens.astype(jnp.int32).reshape(NW, NCH_W, CHUNK)
    out = _emb_call(idx, token_table.astype(jnp.float32),
                    position_table.astype(jnp.float32))
    return out.reshape(BATCH, NUM_POS, EMBED)


# speedup vs baseline: 1.1849x; 1.1849x over previous
import functools

import jax
import jax.numpy as jnp
from jax import lax
from jax.experimental import pallas as pl
from jax.experimental.pallas import tpu as pltpu
from jax.experimental.pallas import tpu_sc as plsc

VOCAB = 49408
NUM_POS = 77
POS_PAD = 80
EMBED = 768
LANE = 128
BATCH = 1024

L = 16
NC = 2
NS = 16
NW = NC * NS
BATCH_PER_W = BATCH // NW
NBUF = 10
CHUNK = 8
TAIL = NUM_POS - 9 * CHUNK
DEPTH = NBUF - 2
NCH_W = BATCH_PER_W * NBUF


def _emb_body(idx_hbm, tok_hbm, pos_hbm, out_hbm, idx_v, pos_v, *rest):
    bufs = rest[:NBUF]
    gsem = rest[NBUF:2 * NBUF]
    ssem = rest[2 * NBUF:3 * NBUF]

    wid = lax.axis_index("s") * NC + lax.axis_index("c")
    lb0 = wid * BATCH_PER_W

    pltpu.sync_copy(idx_hbm.at[pl.ds(lb0 * POS_PAD, BATCH_PER_W * POS_PAD)],
                    idx_v)
    pltpu.sync_copy(pos_hbm, pos_v)

    def gather(lb, j):
        src = tok_hbm.at[idx_v.at[pl.ds(lb * POS_PAD + j * CHUNK, CHUNK)]]
        return pltpu.make_async_copy(src, bufs[j], gsem[j])

    def tail_copies(lb):
        b9 = bufs[NBUF - 1]
        yield pltpu.make_async_copy(
            b9.at[pl.ds(0, TAIL)],
            out_hbm.at[lb0 + lb, pl.ds(9 * CHUNK, TAIL)],
            ssem[NBUF - 1])

    def scatter_start(lb, j):
        if j < NBUF - 1:
            dst = out_hbm.at[lb0 + lb, pl.ds(j * CHUNK, CHUNK)]
            pltpu.make_async_copy(bufs[j], dst, ssem[j]).start()
        else:
            for cp in tail_copies(lb):
                cp.start()

    def scatter_wait(lb, j):
        if j < NBUF - 1:
            dst = out_hbm.at[lb0 + lb, pl.ds(j * CHUNK, CHUNK)]
            pltpu.make_async_copy(bufs[j], dst, ssem[j]).wait()
        else:
            for cp in tail_copies(lb):
                cp.wait()

    for j in range(DEPTH):
        gather(0, j).start()

    def outer(lb, carry):
        for j in range(NBUF):
            cglob = lb * NBUF + j
            jq = (j + DEPTH) % NBUF
            lbq = lb + (0 if j + DEPTH < NBUF else 1)

            @pl.when(cglob + DEPTH < NCH_W)
            def _():
                @pl.when(cglob >= 2)
                def _():
                    scatter_wait(lbq - 1, jq)
                gather(lbq, jq).start()

            gather(lb, j).wait()

            def row_add(r, carry2):
                for k in range(EMBED // L):
                    sl = pl.ds(k * L, L)
                    plsc.addupdate(bufs[j].at[r, sl],
                                   pos_v[j * CHUNK + r, sl])
                return carry2

            nrows = CHUNK if j < NBUF - 1 else TAIL
            lax.fori_loop(0, nrows, row_add, 0, unroll=False)

            scatter_start(lb, j)
        return carry

    lax.fori_loop(0, BATCH_PER_W, outer, 0, unroll=False)

    for j in range(NBUF):
        scatter_wait(BATCH_PER_W - 1, j)


@jax.jit
def _emb_call(idx_pad, token_table, position_table):
    info = plsc.get_sparse_core_info()
    assert info.num_cores == NC and info.num_subcores == NS

    mesh = plsc.VectorSubcoreMesh(core_axis_name="c", subcore_axis_name="s",
                                  num_cores=NC)
    run = functools.partial(
        pl.kernel,
        mesh=mesh,
        out_type=jax.ShapeDtypeStruct((BATCH, NUM_POS, EMBED), jnp.float32),
        scratch_types=(
            [pltpu.VMEM((BATCH_PER_W * POS_PAD,), jnp.int32),
             pltpu.VMEM((NUM_POS, EMBED), jnp.float32)]
            + [pltpu.VMEM((CHUNK, EMBED), jnp.float32)] * NBUF
            + [pltpu.SemaphoreType.DMA] * (2 * NBUF)
        ),
    )(_emb_body)
    return run(idx_pad, token_table, position_table)


def kernel(input_tokens, token_table, position_table):
    idx = input_tokens.astype(jnp.int32)
    idx_pad = jnp.pad(idx, ((0, 0), (0, POS_PAD - NUM_POS))).reshape(-1)
    return _emb_call(idx_pad, token_table.astype(jnp.float32),
                     position_table.astype(jnp.float32))
